# baseline (speedup 1.0000x reference)
import functools

import jax
import jax.numpy as jnp
from jax import lax
from jax.experimental import pallas as pl
from jax.experimental.pallas import tpu as pltpu
from jax.experimental.pallas import tpu_sc as plsc

_VOCAB, _SENT, _MEM, _EMB, _BATCH = 100000, 20, 50, 32, 1024
_NW = 32
_BPW = _BATCH // _NW
_NPAIR = _BPW // 2
_IPB = _MEM * _SENT
_STRIDE = 24
_PPB = _MEM * _STRIDE
_PPW = _BPW * _PPB
_IDXPD = 120
_DPB = _PPB // _IDXPD

_SCALE = 4.0 / (_EMB * _SENT)


def _u(s):
    return float((s + 1 - (_SENT + 1) / 2.0) * _SCALE)


_mesh = plsc.VectorSubcoreMesh(core_axis_name="c", subcore_axis_name="s")


_TCH = 160
_NCH = _VOCAB // _TCH
_TSLOT = 20
_TOW = _TCH * _EMB


@functools.partial(
    pl.kernel,
    mesh=_mesh,
    out_type=(
        jax.ShapeDtypeStruct((_BATCH * _PPB,), jnp.int32),
        jax.ShapeDtypeStruct((_VOCAB * _EMB,), jnp.float32),
    ),
    scratch_types=[
        pltpu.VMEM((_MEM, _SENT), jnp.int32),
        pltpu.VMEM((_MEM, _SENT), jnp.int32),
        pltpu.VMEM((_PPW,), jnp.int32),
        pltpu.VMEM((_TCH, _EMB), jnp.float32),
        pltpu.VMEM((_TCH, _EMB), jnp.float32),
        pltpu.VMEM((_TOW,), jnp.float32),
        pltpu.VMEM((_TOW,), jnp.float32),
        pltpu.SemaphoreType.DMA,
        pltpu.SemaphoreType.DMA,
        pltpu.SemaphoreType.DMA,
        pltpu.SemaphoreType.DMA,
        pltpu.SemaphoreType.DMA,
        pltpu.SemaphoreType.DMA,
    ],
)
def _repack_kernel(x_hbm, tab_in, out_hbm, tabf_out,
                   xb0, xb1, flat_v, tb0, tb1, to0, to1,
                   sem0, sem1, tin0, tin1, tos0, tos1):
    wid = lax.axis_index("s") * 2 + lax.axis_index("c")
    base_b = wid * _BPW

    def fire(b, buf, sem):
        pltpu.async_copy(x_hbm.at[base_b + b], buf, sem)

    def wait(buf, sem):
        pltpu.make_async_copy(x_hbm.at[base_b], buf, sem).wait()

    def repack(b, buf):
        for m in range(_MEM):
            base = b * _PPB + m * _STRIDE
            flat_v[pl.ds(base + 8, 16)] = buf[m, pl.ds(4, 16)]
            flat_v[pl.ds(base, 16)] = buf[m, pl.ds(0, 16)]

    fire(0, xb0, sem0)

    def pair(p, carry):
        ba = 2 * p
        bb = ba + 1
        fire(bb, xb1, sem1)
        wait(xb0, sem0)
        repack(ba, xb0)

        @pl.when(p <= _NPAIR - 2)
        def _():
            fire(bb + 1, xb0, sem0)

        wait(xb1, sem1)
        repack(bb, xb1)
        return carry

    lax.fori_loop(0, _NPAIR, pair, 0)
    pltpu.sync_copy(flat_v, out_hbm.at[pl.ds(wid * _PPW, _PPW)])


    def tcond(k):
        return wid + 32 * k < _NCH

    def fire_tin(k, buf, sem):
        @pl.when(tcond(k))
        def _():
            c = wid + 32 * k
            pltpu.async_copy(tab_in.at[pl.ds(c * _TCH, _TCH)], buf, sem)

    def wait_tin(k, buf, sem):
        @pl.when(tcond(k))
        def _():
            pltpu.make_async_copy(
                tab_in.at[pl.ds(0, _TCH)], buf, sem).wait()

    def depad(k, buf, to):
        @pl.when(tcond(k))
        def _():
            def row(r, c2):
                to[pl.ds(r * _EMB, 16)] = buf[r, pl.ds(0, 16)]
                to[pl.ds(r * _EMB + 16, 16)] = buf[r, pl.ds(16, 16)]
                return c2

            lax.fori_loop(0, _TCH, row, 0)

    def fire_tout(k, to, sem):
        @pl.when(tcond(k))
        def _():
            c = wid + 32 * k
            pltpu.async_copy(to, tabf_out.at[pl.ds(c * _TOW, _TOW)], sem)

    def wait_tout(k, to, sem):
        @pl.when(tcond(k))
        def _():
            pltpu.make_async_copy(
                to, tabf_out.at[pl.ds(0, _TOW)], sem).wait()

    fire_tin(0, tb0, tin0)

    def tpair(kp, carry):
        k0 = 2 * kp
        k1 = k0 + 1
        fire_tin(k1, tb1, tin1)
        wait_tin(k0, tb0, tin0)

        @pl.when(kp >= 1)
        def _():
            wait_tout(k0 - 2, to0, tos0)

        depad(k0, tb0, to0)
        fire_tout(k0, to0, tos0)

        @pl.when(kp <= _TSLOT // 2 - 2)
        def _():
            fire_tin(k0 + 2, tb0, tin0)

        wait_tin(k1, tb1, tin1)

        @pl.when(kp >= 1)
        def _():
            wait_tout(k1 - 2, to1, tos1)

        depad(k1, tb1, to1)
        fire_tout(k1, to1, tos1)
        return carry

    lax.fori_loop(0, _TSLOT // 2, tpair, 0)
    wait_tout(_TSLOT - 2, to0, tos0)
    wait_tout(_TSLOT - 1, to1, tos1)


@functools.partial(
    pl.kernel,
    mesh=_mesh,
    compiler_params=pltpu.CompilerParams(use_tc_tiling_on_sc=False),
    out_type=jax.ShapeDtypeStruct((_BATCH, _MEM, _EMB), jnp.float32),
    scratch_types=[
        pltpu.VMEM((_PPW,), jnp.int32),
        pltpu.VMEM((_PPB, _EMB), jnp.float32),
        pltpu.VMEM((_PPB, _EMB), jnp.float32),
        pltpu.VMEM((_MEM, _EMB), jnp.float32),
        pltpu.VMEM((_MEM, _EMB), jnp.float32),
        pltpu.VMEM((_MEM, _EMB), jnp.float32),
        pltpu.SemaphoreType.DMA,
        pltpu.SemaphoreType.DMA,
        pltpu.SemaphoreType.DMA,
        pltpu.SemaphoreType.DMA,
    ],
)
def _emb_kernel(xp_hbm, tab_hbm, temp_hbm, out_hbm,
                idx_v, rows0, rows1, outb0, outb1, temp_v,
                gsem0, gsem1, osem0, osem1):
    wid = lax.axis_index("s") * 2 + lax.axis_index("c")
    base_b = wid * _BPW
    pltpu.sync_copy(xp_hbm.at[pl.ds(wid * _PPW, _PPW)], idx_v)
    pltpu.sync_copy(temp_hbm, temp_v)

    v0 = lax.iota(jnp.int32, 16).astype(jnp.float32) - 15.5
    v1 = v0 + 16.0

    def fire(b, rows_buf, gsem):
        base = b * _PPB
        for j in range(_DPB):
            pltpu.async_copy(
                tab_hbm.at[idx_v.at[pl.ds(base + j * _IDXPD, _IDXPD)]],
                rows_buf.at[pl.ds(j * _IDXPD, _IDXPD)],
                gsem,
            )

    def wait_gather(rows_buf, gsem):
        pltpu.make_async_copy(
            tab_hbm.at[pl.ds(0, _PPB)], rows_buf, gsem).wait()

    def fire_out(b, outb, osem):
        pltpu.async_copy(outb, out_hbm.at[base_b + b], osem)

    def wait_out(outb, osem):
        pltpu.make_async_copy(outb, out_hbm.at[base_b], osem).wait()

    def one_seg(rows_buf, outb, i):
        r = i * _STRIDE
        acc0 = rows_buf[r, pl.ds(0, 16)] * _u(0)
        acc1 = rows_buf[r, pl.ds(16, 16)] * _u(0)
        for s in range(1, _SENT - 1):
            o = r + s if s < 16 else r + s + 4
            acc0 = acc0 + rows_buf[o, pl.ds(0, 16)] * _u(s)
            acc1 = acc1 + rows_buf[o, pl.ds(16, 16)] * _u(s)
        rl = r + _SENT - 1 + 4
        last0 = rows_buf[rl, pl.ds(0, 16)] + temp_v[i, pl.ds(0, 16)]
        last1 = rows_buf[rl, pl.ds(16, 16)] + temp_v[i, pl.ds(16, 16)]
        outb[i, pl.ds(0, 16)] = acc0 * v0 + last0
        outb[i, pl.ds(16, 16)] = acc1 * v1 + last1

    def compute(rows_buf, outb):
        def seg2(k, c2):
            i = k * 2
            one_seg(rows_buf, outb, i)
            one_seg(rows_buf, outb, i + 1)
            return c2

        lax.fori_loop(0, _MEM // 2, seg2, 0)

    fire(0, rows0, gsem0)

    def pair(p, carry):
        ba = 2 * p
        bb = ba + 1
        fire(bb, rows1, gsem1)
        wait_gather(rows0, gsem0)

        @pl.when(p >= 1)
        def _():
            wait_out(outb0, osem0)

        compute(rows0, outb0)
        fire_out(ba, outb0, osem0)

        @pl.when(p <= _NPAIR - 2)
        def _():
            fire(bb + 1, rows0, gsem0)

        wait_gather(rows1, gsem1)

        @pl.when(p >= 1)
        def _():
            wait_out(outb1, osem1)

        compute(rows1, outb1)
        fire_out(bb, outb1, osem1)
        return carry

    lax.fori_loop(0, _NPAIR, pair, 0)
    wait_out(outb0, osem0)
    wait_out(outb1, osem1)


def kernel(x, emb_table, temporal_table):
    x_pad, tab_flat = _repack_kernel(x.astype(jnp.int32), emb_table)
    tab2 = tab_flat.reshape(_VOCAB, _EMB)
    return _emb_kernel(x_pad, tab2, temporal_table)


# speedup vs baseline: 1.2123x; 1.2123x over previous
import functools

import jax
import jax.numpy as jnp
from jax import lax
from jax.experimental import pallas as pl
from jax.experimental.pallas import tpu as pltpu
from jax.experimental.pallas import tpu_sc as plsc

_VOCAB, _SENT, _MEM, _EMB, _BATCH = 100000, 20, 50, 32, 1024
_NW = 32
_BPW = _BATCH // _NW
_NPAIR = _BPW // 2
_IPB = _MEM * _SENT
_STRIDE = 24
_PPB = _MEM * _STRIDE
_PPW = _BPW * _PPB
_IDXPD = 120
_DPB = _PPB // _IDXPD

_SCALE = 4.0 / (_EMB * _SENT)


def _u(s):
    return float((s + 1 - (_SENT + 1) / 2.0) * _SCALE)


_mesh = plsc.VectorSubcoreMesh(core_axis_name="c", subcore_axis_name="s")


@functools.partial(
    pl.kernel,
    mesh=_mesh,
    out_type=jax.ShapeDtypeStruct((_BATCH * _PPB,), jnp.int32),
    scratch_types=[
        pltpu.VMEM((_MEM, _SENT), jnp.int32),
        pltpu.VMEM((_MEM, _SENT), jnp.int32),
        pltpu.VMEM((_PPW,), jnp.int32),
        pltpu.SemaphoreType.DMA,
        pltpu.SemaphoreType.DMA,
    ],
)
def _repack_kernel(x_hbm, out_hbm, xb0, xb1, flat_v, sem0, sem1):
    wid = lax.axis_index("s") * 2 + lax.axis_index("c")
    base_b = wid * _BPW

    def fire(b, buf, sem):
        pltpu.async_copy(x_hbm.at[base_b + b], buf, sem)

    def wait(buf, sem):
        pltpu.make_async_copy(x_hbm.at[base_b], buf, sem).wait()

    def repack(b, buf):
        for m in range(_MEM):
            base = b * _PPB + m * _STRIDE
            flat_v[pl.ds(base + 8, 16)] = buf[m, pl.ds(4, 16)]
            flat_v[pl.ds(base, 16)] = buf[m, pl.ds(0, 16)]

    fire(0, xb0, sem0)

    def pair(p, carry):
        ba = 2 * p
        bb = ba + 1
        fire(bb, xb1, sem1)
        wait(xb0, sem0)
        repack(ba, xb0)

        @pl.when(p <= _NPAIR - 2)
        def _():
            fire(bb + 1, xb0, sem0)

        wait(xb1, sem1)
        repack(bb, xb1)
        return carry

    lax.fori_loop(0, _NPAIR, pair, 0)
    pltpu.sync_copy(flat_v, out_hbm.at[pl.ds(wid * _PPW, _PPW)])


@functools.partial(
    pl.kernel,
    mesh=_mesh,
    compiler_params=pltpu.CompilerParams(use_tc_tiling_on_sc=False),
    out_type=jax.ShapeDtypeStruct((_BATCH * _MEM * _EMB,), jnp.float32),
    scratch_types=[
        pltpu.VMEM((_PPW,), jnp.int32),
        pltpu.VMEM((_PPB, _EMB), jnp.float32),
        pltpu.VMEM((_PPB, _EMB), jnp.float32),
        pltpu.VMEM((_MEM * _EMB,), jnp.float32),
        pltpu.VMEM((_MEM * _EMB,), jnp.float32),
        pltpu.VMEM((_MEM, _EMB), jnp.float32),
        pltpu.SemaphoreType.DMA,
        pltpu.SemaphoreType.DMA,
        pltpu.SemaphoreType.DMA,
        pltpu.SemaphoreType.DMA,
    ],
)
def _emb_kernel(xp_hbm, tab_hbm, temp_hbm, out_hbm,
                idx_v, rows0, rows1, outb0, outb1, temp_v,
                gsem0, gsem1, osem0, osem1):
    wid = lax.axis_index("s") * 2 + lax.axis_index("c")
    base_b = wid * _BPW
    pltpu.sync_copy(xp_hbm.at[pl.ds(wid * _PPW, _PPW)], idx_v)
    pltpu.sync_copy(temp_hbm, temp_v)

    v0 = lax.iota(jnp.int32, 16).astype(jnp.float32) - 15.5
    v1 = v0 + 16.0

    def fire(b, rows_buf, gsem):
        base = b * _PPB
        for j in range(_DPB):
            pltpu.async_copy(
                tab_hbm.at[idx_v.at[pl.ds(base + j * _IDXPD, _IDXPD)]],
                rows_buf.at[pl.ds(j * _IDXPD, _IDXPD)],
                gsem,
            )

    def wait_gather(rows_buf, gsem):
        pltpu.make_async_copy(
            tab_hbm.at[pl.ds(0, _PPB)], rows_buf, gsem).wait()

    def fire_out(b, outb, osem):
        pltpu.async_copy(
            outb, out_hbm.at[pl.ds((base_b + b) * _MEM * _EMB, _MEM * _EMB)],
            osem)

    def wait_out(outb, osem):
        pltpu.make_async_copy(
            outb, out_hbm.at[pl.ds(0, _MEM * _EMB)], osem).wait()

    def one_seg(rows_buf, outb, i):
        r = i * _STRIDE
        acc0 = rows_buf[r, pl.ds(0, 16)] * _u(0)
        acc1 = rows_buf[r, pl.ds(16, 16)] * _u(0)
        for s in range(1, _SENT - 1):
            o = r + s if s < 16 else r + s + 4
            acc0 = acc0 + rows_buf[o, pl.ds(0, 16)] * _u(s)
            acc1 = acc1 + rows_buf[o, pl.ds(16, 16)] * _u(s)
        rl = r + _SENT - 1 + 4
        last0 = rows_buf[rl, pl.ds(0, 16)] + temp_v[i, pl.ds(0, 16)]
        last1 = rows_buf[rl, pl.ds(16, 16)] + temp_v[i, pl.ds(16, 16)]
        outb[pl.ds(i * _EMB, 16)] = acc0 * v0 + last0
        outb[pl.ds(i * _EMB + 16, 16)] = acc1 * v1 + last1

    def compute(rows_buf, outb):
        def seg2(k, c2):
            i = k * 2
            one_seg(rows_buf, outb, i)
            one_seg(rows_buf, outb, i + 1)
            return c2

        lax.fori_loop(0, _MEM // 2, seg2, 0)

    fire(0, rows0, gsem0)

    def pair(p, carry):
        ba = 2 * p
        bb = ba + 1
        fire(bb, rows1, gsem1)
        wait_gather(rows0, gsem0)

        @pl.when(p >= 1)
        def _():
            wait_out(outb0, osem0)

        compute(rows0, outb0)
        fire_out(ba, outb0, osem0)

        @pl.when(p <= _NPAIR - 2)
        def _():
            fire(bb + 1, rows0, gsem0)

        wait_gather(rows1, gsem1)

        @pl.when(p >= 1)
        def _():
            wait_out(outb1, osem1)

        compute(rows1, outb1)
        fire_out(bb, outb1, osem1)
        return carry

    lax.fori_loop(0, _NPAIR, pair, 0)
    wait_out(outb0, osem0)
    wait_out(outb1, osem1)


def kernel(x, emb_table, temporal_table):
    x_pad = _repack_kernel(x.astype(jnp.int32))
    out = _emb_kernel(x_pad, emb_table, temporal_table)
    return out.reshape(_BATCH, _MEM, _EMB)


# speedup vs baseline: 1.2157x; 1.0028x over previous
import functools

import jax
import jax.numpy as jnp
from jax import lax
from jax.experimental import pallas as pl
from jax.experimental.pallas import tpu as pltpu
from jax.experimental.pallas import tpu_sc as plsc

_VOCAB, _SENT, _MEM, _EMB, _BATCH = 100000, 20, 50, 32, 1024
_NW = 32
_BPW = _BATCH // _NW
_NPAIR = _BPW // 2
_IPB = _MEM * _SENT
_STRIDE = 24
_PPB = _MEM * _STRIDE
_PPW = _BPW * _PPB
_IDXPD = 120
_DPB = _PPB // _IDXPD

_SCALE = 4.0 / (_EMB * _SENT)


def _u(s):
    return float((s + 1 - (_SENT + 1) / 2.0) * _SCALE)


_mesh = plsc.VectorSubcoreMesh(core_axis_name="c", subcore_axis_name="s")


@functools.partial(
    pl.kernel,
    mesh=_mesh,
    out_type=jax.ShapeDtypeStruct((_BATCH * _PPB,), jnp.int32),
    scratch_types=[
        pltpu.VMEM((_MEM, _SENT), jnp.int32),
        pltpu.VMEM((_MEM, _SENT), jnp.int32),
        pltpu.VMEM((_PPW,), jnp.int32),
        pltpu.SemaphoreType.DMA,
        pltpu.SemaphoreType.DMA,
    ],
)
def _repack_kernel(x_hbm, out_hbm, xb0, xb1, flat_v, sem0, sem1):
    wid = lax.axis_index("s") * 2 + lax.axis_index("c")
    base_b = wid * _BPW

    def fire(b, buf, sem):
        pltpu.async_copy(x_hbm.at[base_b + b], buf, sem)

    def wait(buf, sem):
        pltpu.make_async_copy(x_hbm.at[base_b], buf, sem).wait()

    def repack(b, buf):
        for m in range(_MEM):
            base = b * _PPB + m * _STRIDE
            flat_v[pl.ds(base + 8, 16)] = buf[m, pl.ds(4, 16)]
            flat_v[pl.ds(base, 16)] = buf[m, pl.ds(0, 16)]

    fire(0, xb0, sem0)

    def pair(p, carry):
        ba = 2 * p
        bb = ba + 1
        fire(bb, xb1, sem1)
        wait(xb0, sem0)
        repack(ba, xb0)

        @pl.when(p <= _NPAIR - 2)
        def _():
            fire(bb + 1, xb0, sem0)

        wait(xb1, sem1)
        repack(bb, xb1)
        return carry

    lax.fori_loop(0, _NPAIR, pair, 0)
    pltpu.sync_copy(flat_v, out_hbm.at[pl.ds(wid * _PPW, _PPW)])


@functools.partial(
    pl.kernel,
    mesh=_mesh,
    compiler_params=pltpu.CompilerParams(use_tc_tiling_on_sc=False),
    out_type=jax.ShapeDtypeStruct((_BATCH * _MEM * _EMB,), jnp.float32),
    scratch_types=[
        pltpu.VMEM((_PPW,), jnp.int32),
        pltpu.VMEM((_PPB, _EMB), jnp.float32),
        pltpu.VMEM((_PPB, _EMB), jnp.float32),
        pltpu.VMEM((_MEM * _EMB,), jnp.float32),
        pltpu.VMEM((_MEM * _EMB,), jnp.float32),
        pltpu.VMEM((_MEM, _EMB), jnp.float32),
        pltpu.SemaphoreType.DMA,
        pltpu.SemaphoreType.DMA,
        pltpu.SemaphoreType.DMA,
        pltpu.SemaphoreType.DMA,
    ],
)
def _emb_kernel(xp_hbm, tab_hbm, temp_hbm, out_hbm,
                idx_v, rows0, rows1, outb0, outb1, temp_v,
                gsem0, gsem1, osem0, osem1):
    wid = lax.axis_index("s") * 2 + lax.axis_index("c")
    base_b = wid * _BPW
    pltpu.sync_copy(xp_hbm.at[pl.ds(wid * _PPW, _PPW)], idx_v)
    pltpu.sync_copy(temp_hbm, temp_v)

    v0 = lax.iota(jnp.int32, 16).astype(jnp.float32) - 15.5
    v1 = v0 + 16.0

    def fire(b, rows_buf, gsem):
        base = b * _PPB
        for j in range(_DPB):
            pltpu.async_copy(
                tab_hbm.at[idx_v.at[pl.ds(base + j * _IDXPD, _IDXPD)]],
                rows_buf.at[pl.ds(j * _IDXPD, _IDXPD)],
                gsem,
            )

    def wait_gather(rows_buf, gsem):
        pltpu.make_async_copy(
            tab_hbm.at[pl.ds(0, _PPB)], rows_buf, gsem).wait()

    def fire_out(b, outb, osem):
        pltpu.async_copy(
            outb, out_hbm.at[pl.ds((base_b + b) * _MEM * _EMB, _MEM * _EMB)],
            osem)

    def wait_out(outb, osem):
        pltpu.make_async_copy(
            outb, out_hbm.at[pl.ds(0, _MEM * _EMB)], osem).wait()

    def one_seg(rows_buf, outb, i):
        r = i * _STRIDE
        acc0 = rows_buf[r, pl.ds(0, 16)] * _u(0)
        acc1 = rows_buf[r, pl.ds(16, 16)] * _u(0)
        for s in range(1, _SENT - 1):
            o = r + s if s < 16 else r + s + 4
            acc0 = acc0 + rows_buf[o, pl.ds(0, 16)] * _u(s)
            acc1 = acc1 + rows_buf[o, pl.ds(16, 16)] * _u(s)
        rl = r + _SENT - 1 + 4
        last0 = rows_buf[rl, pl.ds(0, 16)] + temp_v[i, pl.ds(0, 16)]
        last1 = rows_buf[rl, pl.ds(16, 16)] + temp_v[i, pl.ds(16, 16)]
        outb[pl.ds(i * _EMB, 16)] = acc0 * v0 + last0
        outb[pl.ds(i * _EMB + 16, 16)] = acc1 * v1 + last1

    def compute(rows_buf, outb):
        def seg5(k, c2):
            i = k * 5
            for d in range(5):
                one_seg(rows_buf, outb, i + d)
            return c2

        lax.fori_loop(0, _MEM // 5, seg5, 0)

    fire(0, rows0, gsem0)

    def pair(p, carry):
        ba = 2 * p
        bb = ba + 1
        fire(bb, rows1, gsem1)
        wait_gather(rows0, gsem0)

        @pl.when(p >= 1)
        def _():
            wait_out(outb0, osem0)

        compute(rows0, outb0)
        fire_out(ba, outb0, osem0)

        @pl.when(p <= _NPAIR - 2)
        def _():
            fire(bb + 1, rows0, gsem0)

        wait_gather(rows1, gsem1)

        @pl.when(p >= 1)
        def _():
            wait_out(outb1, osem1)

        compute(rows1, outb1)
        fire_out(bb, outb1, osem1)
        return carry

    lax.fori_loop(0, _NPAIR, pair, 0)
    wait_out(outb0, osem0)
    wait_out(outb1, osem1)


def kernel(x, emb_table, temporal_table):
    x_pad = _repack_kernel(x.astype(jnp.int32))
    out = _emb_kernel(x_pad, emb_table, temporal_table)
    return out.reshape(_BATCH, _MEM, _EMB)
